# baseline (speedup 1.0000x reference)
import functools

import jax
import jax.numpy as jnp
from jax import lax
from jax.experimental import pallas as pl
from jax.experimental.pallas import tpu as pltpu
from jax.experimental.pallas import tpu_sc as plsc

NC, NS, L = 2, 16, 16
NW = NC * NS
C = 80


@functools.lru_cache(maxsize=None)
def _build(B, V, D):
    BPW = B // NW
    NCH = BPW // C
    assert BPW * NW == B and NCH * C == BPW and NCH % 2 == 0
    assert C % L == 0 and D % L == 0

    mesh = plsc.VectorSubcoreMesh(core_axis_name="c", subcore_axis_name="s")

    @functools.partial(
        pl.kernel,
        out_type=jax.ShapeDtypeStruct((B * D,), jnp.float32),
        mesh=mesh,
        compiler_params=pltpu.CompilerParams(needs_layout_passes=False),
        scratch_types=[
            pltpu.VMEM((BPW,), jnp.int32),
            pltpu.VMEM((V * D,), jnp.float32),
            pltpu.VMEM((C * D,), jnp.float32),
            pltpu.VMEM((C * D,), jnp.float32),
            pltpu.SemaphoreType.DMA,
            pltpu.SemaphoreType.DMA,
        ],
    )
    def emb(idx_hbm, table_hbm, out_hbm, idx_v, table_v, rows0, rows1, o0, o1):
        rows = (rows0, rows1)
        osem = (o0, o1)
        wid = lax.axis_index("s") * NC + lax.axis_index("c")
        base = wid * BPW
        pltpu.sync_copy(idx_hbm.at[pl.ds(base, BPW)], idx_v)
        pltpu.sync_copy(table_hbm, table_v)

        def wait_o(b):
            pltpu.make_async_copy(rows[b], out_hbm.at[pl.ds(0, C * D)],
                                  osem[b]).wait()

        def step(c, b):
            @pl.when(c >= 2)
            def _():
                wait_o(b)

            for g in range(C // L):
                idx16 = idx_v[pl.ds(c * C + g * L, L)]
                rowbase = idx16 * D
                posbase = (lax.iota(jnp.int32, L) + g * L) * D

                def colbody(cb, carry, rowbase=rowbase, posbase=posbase, b=b):
                    rc = rowbase + cb * L
                    pc = posbase + cb * L
                    for u in range(L):
                        v = plsc.load_gather(table_v, [rc + u])
                        plsc.store_scatter(rows[b], [pc + u], v)
                    return carry

                lax.fori_loop(0, D // L, colbody, 0)

            pltpu.async_copy(rows[b],
                             out_hbm.at[pl.ds((base + c * C) * D, C * D)],
                             osem[b])

        def body(i, carry):
            step(2 * i, 0)
            step(2 * i + 1, 1)
            return carry

        lax.fori_loop(0, NCH // 2, body, 0)
        wait_o(0)
        wait_o(1)

    return emb


def kernel(word_seq, embedding_table):
    s, t = word_seq.shape
    b = s * t
    v, d = embedding_table.shape
    idx = word_seq.reshape(b).astype(jnp.int32)
    table = embedding_table.astype(jnp.float32).reshape(v * d)
    out = _build(b, v, d)(idx, table)
    return out.reshape(s, t, d)


# speedup vs baseline: 3.6506x; 3.6506x over previous
import functools

import jax
import jax.numpy as jnp
from jax import lax
from jax.experimental import pallas as pl
from jax.experimental.pallas import tpu as pltpu
from jax.experimental.pallas import tpu_sc as plsc

NC, NS, L = 2, 16, 16
NW = NC * NS
C = 80


@functools.lru_cache(maxsize=None)
def _build(B, V, D):
    BPW = B // NW
    NCH = BPW // C
    assert BPW * NW == B and NCH * C == BPW and NCH % 2 == 0
    assert C % L == 0 and D % L == 0

    mesh = plsc.VectorSubcoreMesh(core_axis_name="c", subcore_axis_name="s")

    @functools.partial(
        pl.kernel,
        out_type=jax.ShapeDtypeStruct((B * D,), jnp.float32),
        mesh=mesh,
        compiler_params=pltpu.CompilerParams(needs_layout_passes=False),
        scratch_types=[
            pltpu.VMEM((BPW,), jnp.int32),
            pltpu.VMEM((V * D,), jnp.float32),
            pltpu.VMEM((C * D,), jnp.float32),
            pltpu.VMEM((C * D,), jnp.float32),
            pltpu.SemaphoreType.DMA,
            pltpu.SemaphoreType.DMA,
        ],
    )
    def emb(idx_hbm, table_hbm, out_hbm, idx_v, table_v, rows0, rows1, o0, o1):
        rows = (rows0, rows1)
        osem = (o0, o1)
        wid = lax.axis_index("s") * NC + lax.axis_index("c")
        base = wid * BPW
        pltpu.sync_copy(idx_hbm.at[pl.ds(base, BPW)], idx_v)
        pltpu.sync_copy(table_hbm, table_v)

        def wait_o(b):
            pltpu.make_async_copy(rows[b], out_hbm.at[pl.ds(0, C * D)],
                                  osem[b]).wait()

        def step(c, b):
            @pl.when(c >= 2)
            def _():
                wait_o(b)

            def gbody(g, carry, b=b):
                idx16 = idx_v[pl.ds(c * C + g * L, L)]
                rowpos = (g * L) * D
                for u in range(L):
                    tb = idx16[u] * D
                    for j in range(D // L):
                        rows[b][pl.ds(rowpos + u * D + j * L, L)] = (
                            table_v[pl.ds(tb + j * L, L)])
                return carry

            lax.fori_loop(0, C // L, gbody, 0)

            pltpu.async_copy(rows[b],
                             out_hbm.at[pl.ds((base + c * C) * D, C * D)],
                             osem[b])

        def body(i, carry):
            step(2 * i, 0)
            step(2 * i + 1, 1)
            return carry

        lax.fori_loop(0, NCH // 2, body, 0)
        wait_o(0)
        wait_o(1)

    return emb


def kernel(word_seq, embedding_table):
    s, t = word_seq.shape
    b = s * t
    v, d = embedding_table.shape
    idx = word_seq.reshape(b).astype(jnp.int32)
    table = embedding_table.astype(jnp.float32).reshape(v * d)
    out = _build(b, v, d)(idx, table)
    return out.reshape(s, t, d)


# speedup vs baseline: 7.3114x; 2.0028x over previous
import functools

import jax
import jax.numpy as jnp
from jax import lax
from jax.experimental import pallas as pl
from jax.experimental.pallas import tpu as pltpu
from jax.experimental.pallas import tpu_sc as plsc

NC, NS, L = 2, 16, 16
NW = NC * NS
C = 80


@functools.lru_cache(maxsize=None)
def _build(B, V, D):
    BPW = B // NW
    NCH = BPW // C
    assert BPW * NW == B and NCH * C == BPW and NCH % 2 == 0
    assert C % L == 0 and D % L == 0

    mesh = plsc.VectorSubcoreMesh(core_axis_name="c", subcore_axis_name="s")

    @functools.partial(
        pl.kernel,
        out_type=jax.ShapeDtypeStruct((B * D,), jnp.float32),
        mesh=mesh,
        compiler_params=pltpu.CompilerParams(needs_layout_passes=False),
        scratch_types=[
            pltpu.VMEM((BPW,), jnp.int32),
            pltpu.VMEM((V * D,), jnp.float32),
            pltpu.VMEM((C * D,), jnp.float32),
            pltpu.VMEM((C * D,), jnp.float32),
            pltpu.SemaphoreType.DMA,
            pltpu.SemaphoreType.DMA,
        ],
    )
    def emb(idx_hbm, table_hbm, out_hbm, idx_v, table_v, rows0, rows1, o0, o1):
        rows = (rows0, rows1)
        osem = (o0, o1)
        wid = lax.axis_index("s") * NC + lax.axis_index("c")
        base = wid * BPW
        pltpu.sync_copy(idx_hbm.at[pl.ds(base, BPW)], idx_v)
        pltpu.sync_copy(table_hbm, table_v)

        def wait_o(b):
            pltpu.make_async_copy(rows[b], out_hbm.at[pl.ds(0, C * D)],
                                  osem[b]).wait()

        def step(c, b):
            @pl.when(c >= 2)
            def _():
                wait_o(b)

            def gbody(g, carry, b=b):
                idx16 = idx_v[pl.ds(c * C + g * L, L)]
                rowpos = (g * L) * D
                for u in range(L):
                    tb = idx16[u] * D
                    dst = rowpos + u * D

                    @plsc.parallel_loop(0, D // L, unroll=D // L)
                    def jbody(j, tb=tb, dst=dst, b=b):
                        rows[b][pl.ds(dst + j * L, L)] = (
                            table_v[pl.ds(tb + j * L, L)])
                return carry

            lax.fori_loop(0, C // L, gbody, 0)

            pltpu.async_copy(rows[b],
                             out_hbm.at[pl.ds((base + c * C) * D, C * D)],
                             osem[b])

        def body(i, carry):
            step(2 * i, 0)
            step(2 * i + 1, 1)
            return carry

        lax.fori_loop(0, NCH // 2, body, 0)
        wait_o(0)
        wait_o(1)

    return emb


def kernel(word_seq, embedding_table):
    s, t = word_seq.shape
    b = s * t
    v, d = embedding_table.shape
    idx = word_seq.reshape(b).astype(jnp.int32)
    table = embedding_table.astype(jnp.float32).reshape(v * d)
    out = _build(b, v, d)(idx, table)
    return out.reshape(s, t, d)
